# speedup vs baseline: 1.7551x; 1.7551x over previous
import jax
import jax.numpy as jnp
from jax import lax
from jax.experimental import pallas as pl
from jax.experimental.pallas import tpu as pltpu
from jax.experimental.pallas import tpu_sc as plsc

NUM_WORKERS = 32
CHUNK = 128
CHUNKS_PER_WORKER = 4


def _gather_kernel(idx_hbm, table_hbm, out_hbm, idx_v, rows_v, sem):
    wid = lax.axis_index("s") * 2 + lax.axis_index("c")
    base = wid * CHUNKS_PER_WORKER
    pltpu.sync_copy(idx_hbm.at[pl.ds(base, CHUNKS_PER_WORKER)], idx_v)
    copies = [
        pltpu.async_copy(table_hbm.at[idx_v.at[j]], rows_v.at[j], sem)
        for j in range(CHUNKS_PER_WORKER)
    ]
    for c in copies:
        c.wait()
    pltpu.sync_copy(rows_v, out_hbm.at[pl.ds(base, CHUNKS_PER_WORKER)])


def kernel(indices, table):
    batch = indices.shape[0]
    dims = table.shape[1]
    idx2d = indices.astype(jnp.int32).reshape(batch // CHUNK, CHUNK)
    mesh = plsc.VectorSubcoreMesh(core_axis_name="c", subcore_axis_name="s")
    out = pl.kernel(
        _gather_kernel,
        out_type=jax.ShapeDtypeStruct((batch // CHUNK, CHUNK, dims), jnp.float32),
        mesh=mesh,
        scratch_types=[
            pltpu.VMEM((CHUNKS_PER_WORKER, CHUNK), jnp.int32),
            pltpu.VMEM((CHUNKS_PER_WORKER, CHUNK, dims), jnp.float32),
            pltpu.SemaphoreType.DMA,
        ],
    )(idx2d, table)
    return out.reshape(batch, dims)


# speedup vs baseline: 2.9032x; 1.6542x over previous
import jax
import jax.numpy as jnp
from jax import lax
from jax.experimental import pallas as pl
from jax.experimental.pallas import tpu as pltpu
from jax.experimental.pallas import tpu_sc as plsc

NUM_WORKERS = 32
CHUNK = 128
CHUNKS_PER_WORKER = 4


def _gather_kernel(idx_hbm, table_hbm, out_hbm, idx_v, rows_v, tbl_sh,
                   sem_g, sem_o):
    sid = lax.axis_index("s")
    wid = sid * 2 + lax.axis_index("c")
    base = wid * CHUNKS_PER_WORKER
    idx_cp = pltpu.async_copy(idx_hbm.at[pl.ds(base, CHUNKS_PER_WORKER)],
                              idx_v, sem_g)

    @pl.when(sid == 0)
    def _stage_table():
        pltpu.sync_copy(table_hbm, tbl_sh)

    plsc.subcore_barrier()
    idx_cp.wait()

    gathers = [
        pltpu.async_copy(tbl_sh.at[idx_v.at[j]], rows_v.at[j], sem_g)
        for j in range(CHUNKS_PER_WORKER)
    ]
    writes = []
    for j in range(CHUNKS_PER_WORKER):
        gathers[j].wait()
        writes.append(
            pltpu.async_copy(rows_v.at[j], out_hbm.at[base + j], sem_o))
    for c in writes:
        c.wait()


def kernel(indices, table):
    batch = indices.shape[0]
    keys, dims = table.shape
    idx2d = indices.astype(jnp.int32).reshape(batch // CHUNK, CHUNK)
    mesh = plsc.VectorSubcoreMesh(core_axis_name="c", subcore_axis_name="s")
    out = pl.kernel(
        _gather_kernel,
        out_type=jax.ShapeDtypeStruct((batch // CHUNK, CHUNK, dims), jnp.float32),
        mesh=mesh,
        scratch_types=[
            pltpu.VMEM((CHUNKS_PER_WORKER, CHUNK), jnp.int32),
            pltpu.VMEM((CHUNKS_PER_WORKER, CHUNK, dims), jnp.float32),
            pltpu.VMEM_SHARED((keys, dims), jnp.float32),
            pltpu.SemaphoreType.DMA,
            pltpu.SemaphoreType.DMA,
        ],
    )(idx2d, table)
    return out.reshape(batch, dims)
